# baseline (speedup 1.0000x reference)
import functools

import jax
import jax.numpy as jnp
from jax import lax
from jax.experimental import pallas as pl
from jax.experimental.pallas import tpu as pltpu
from jax.experimental.pallas import tpu_sc as plsc

B = 16384
D = 256
S = 8
K = 256
DS = D // S

TB = 2048

_HI = lax.Precision.HIGHEST


def _dot3(a, b):
    a_hi = a.astype(jnp.bfloat16)
    a_lo = (a - a_hi.astype(jnp.float32)).astype(jnp.bfloat16)
    b_hi = b.astype(jnp.bfloat16)
    b_lo = (b - b_hi.astype(jnp.float32)).astype(jnp.bfloat16)
    dims = (((1,), (1,)), ((), ()))
    dot = lambda u, v: lax.dot_general(u, v, dims,
                                       preferred_element_type=jnp.float32)
    return dot(a_hi, b_hi) + dot(a_hi, b_lo) + dot(a_lo, b_hi)


def _pq_codes_body(x_ref, cb_ref, codes_ref):
    x = x_ref[...]
    code_rows = []
    for s in range(S):
        cb = cb_ref[s]
        xs = x[:, s * DS:(s + 1) * DS]
        g = _dot3(cb, xs)
        cn = jnp.sum(cb * cb, axis=1, keepdims=True)
        d = cn - 2.0 * g
        idx = jnp.argmin(d, axis=0).astype(jnp.int32).reshape(1, TB)
        code_rows.append(idx)
    codes_ref[...] = jnp.concatenate(code_rows, axis=0)


_INFO = plsc.get_sparse_core_info()
_NC = _INFO.num_cores
_NS = _INFO.num_subcores
_NW = _NC * _NS
BH = B // 2
_BS = BH * S
_RPW = _BS // _NW
_CH = 1024
_NCHUNK = _RPW // _CH
_L = _INFO.num_lanes


@functools.partial(
    pl.kernel,
    mesh=plsc.VectorSubcoreMesh(core_axis_name="c", subcore_axis_name="s"),
    out_type=jax.ShapeDtypeStruct((_BS, DS), jnp.float32),
    scratch_types=[
        [pltpu.VMEM((_CH,), jnp.int32) for _ in range(_NCHUNK)],
        [pltpu.VMEM((_CH,), jnp.int32) for _ in range(_NCHUNK)],
        [pltpu.VMEM((_CH, DS), jnp.float32) for _ in range(2)],
        [pltpu.SemaphoreType.DMA for _ in range(2)],
        [pltpu.SemaphoreType.DMA for _ in range(2)],
    ],
    compiler_params=pltpu.CompilerParams(use_tc_tiling_on_sc=False),
)
def _sc_recon(codes_hbm, table_hbm, out_hbm, idx_v, dst_v, rows_v, gsem, ssem):
    wid = lax.axis_index("s") * _NC + lax.axis_index("c")
    base = wid * _RPW
    s_id = base // BH
    b0 = base - s_id * BH
    soff = s_id * K

    for j in range(_NCHUNK):
        pltpu.sync_copy(codes_hbm.at[pl.ds(base + j * _CH, _CH)], idx_v[j])

    lane = lax.iota(jnp.int32, _L)
    for j in range(_NCHUNK):
        for v in range(_CH // _L):
            sl = pl.ds(v * _L, _L)
            idx_v[j][sl] = idx_v[j][sl] + soff
            bvec = b0 + j * _CH + v * _L + lane
            dst_v[j][sl] = bvec * S + s_id

    gat = [None] * _NCHUNK
    sto = [None] * _NCHUNK

    def start_gather(j):
        gat[j] = pltpu.async_copy(
            table_hbm.at[idx_v[j]], rows_v[j % 2], gsem[j % 2])

    def start_store(j):
        sto[j] = pltpu.async_copy(
            rows_v[j % 2], out_hbm.at[dst_v[j]], ssem[j % 2])

    start_gather(0)
    if _NCHUNK > 1:
        start_gather(1)
    for j in range(_NCHUNK):
        gat[j].wait()
        start_store(j)
        if j + 2 < _NCHUNK:
            sto[j].wait()
            start_gather(j + 2)
    for j in range(max(0, _NCHUNK - 2), _NCHUNK):
        sto[j].wait()


@jax.jit
def kernel(test_embeds, subcodebooks):
    table = subcodebooks.reshape(S * K, DS)
    grid = (BH // TB,)
    codes_halves = []
    recon_halves = []
    for h in range(2):
        off = h * (BH // TB)
        codes_t = pl.pallas_call(
            _pq_codes_body,
            grid=grid,
            in_specs=[
                pl.BlockSpec((TB, D), lambda i, off=off: (off + i, 0)),
                pl.BlockSpec((S, K, DS), lambda i: (0, 0, 0)),
            ],
            out_specs=pl.BlockSpec((S, TB), lambda i: (0, i)),
            out_shape=jax.ShapeDtypeStruct((S, BH), jnp.int32),
        )(test_embeds, subcodebooks)
        rows = _sc_recon(codes_t.reshape(S * BH), table)
        codes_halves.append(codes_t)
        recon_halves.append(rows.reshape(BH, D))
    return (jnp.concatenate(codes_halves, axis=1).T,
            jnp.concatenate(recon_halves, axis=0))
